# baseline (speedup 1.0000x reference)
import functools

import jax
import jax.numpy as jnp
from jax import lax
from jax.experimental import pallas as pl
from jax.experimental.pallas import tpu as pltpu
from jax.experimental.pallas import tpu_sc as plsc

B = 4096
U, I, C = 8, 6, 3
D = 32
N = U * I * C
ROW = 3 * D
NC, NS = 2, 16
NW = NC * NS
BPW = B // NW
L = 16


def _cross_body(u_hbm, i_hbm, c_hbm, out_hbm, u_v, i_v, c_v, o_v):
    cid = lax.axis_index("c")
    sid = lax.axis_index("s")
    wid = sid * NC + cid
    base = wid * BPW

    pltpu.sync_copy(u_hbm.at[pl.ds(base * U * D, BPW * U * D)], u_v)
    pltpu.sync_copy(i_hbm.at[pl.ds(base * I * D, BPW * I * D)], i_v)
    pltpu.sync_copy(c_hbm.at[pl.ds(base * C * D, BPW * C * D)], c_v)

    def per_row(r, carry):
        def per_cross(n, carry2):
            i = n // (I * C)
            j = (n // C) % I
            k = n % C
            for t in range(D // L):
                o_v[pl.ds(n * ROW + t * L, L)] = (
                    u_v[pl.ds((r * U + i) * D + t * L, L)])
                o_v[pl.ds(n * ROW + D + t * L, L)] = (
                    i_v[pl.ds((r * I + j) * D + t * L, L)])
                o_v[pl.ds(n * ROW + 2 * D + t * L, L)] = (
                    c_v[pl.ds((r * C + k) * D + t * L, L)])
            return carry2

        lax.fori_loop(0, N, per_cross, 0)
        pltpu.sync_copy(o_v, out_hbm.at[pl.ds((base + r) * N * ROW, N * ROW)])
        return carry

    lax.fori_loop(0, BPW, per_row, 0)


_cross_call = functools.partial(
    pl.kernel,
    out_type=jax.ShapeDtypeStruct((B * N * ROW,), jnp.float32),
    mesh=plsc.VectorSubcoreMesh(
        core_axis_name="c", subcore_axis_name="s",
        num_cores=NC, num_subcores=NS),
    scratch_types=[
        pltpu.VMEM((BPW * U * D,), jnp.float32),
        pltpu.VMEM((BPW * I * D,), jnp.float32),
        pltpu.VMEM((BPW * C * D,), jnp.float32),
        pltpu.VMEM((N * ROW,), jnp.float32),
    ],
)(_cross_body)


@jax.jit
def kernel(x_user, x_item, x_context):
    out = _cross_call(
        x_user.reshape(B * U * D),
        x_item.reshape(B * I * D),
        x_context.reshape(B * C * D),
    )
    return out.reshape(B, N, ROW)


# speedup vs baseline: 1.4427x; 1.4427x over previous
import functools

import jax
import jax.numpy as jnp
from jax import lax
from jax.experimental import pallas as pl
from jax.experimental.pallas import tpu as pltpu
from jax.experimental.pallas import tpu_sc as plsc

B = 4096
U, I, C = 8, 6, 3
D = 32
N = U * I * C
ROW = 3 * D
NR = N * ROW
NC, NS = 2, 16
NW = NC * NS
BPW = B // NW
L = 16


def _build_row(u_v, i_v, c_v, o_v, r):
    us = [u_v[pl.ds(r * U * D + t * L, L)] for t in range(U * D // L)]
    it = [i_v[pl.ds(r * I * D + t * L, L)] for t in range(I * D // L)]
    cs = [c_v[pl.ds(r * C * D + t * L, L)] for t in range(C * D // L)]
    for i in range(U):
        for j in range(I):
            for k in range(C):
                n = (i * I + j) * C + k
                o_v[pl.ds(n * ROW + 0 * L, L)] = us[2 * i]
                o_v[pl.ds(n * ROW + 1 * L, L)] = us[2 * i + 1]
                o_v[pl.ds(n * ROW + 2 * L, L)] = it[2 * j]
                o_v[pl.ds(n * ROW + 3 * L, L)] = it[2 * j + 1]
                o_v[pl.ds(n * ROW + 4 * L, L)] = cs[2 * k]
                o_v[pl.ds(n * ROW + 5 * L, L)] = cs[2 * k + 1]


def _cross_body(u_hbm, i_hbm, c_hbm, out_hbm,
                u_v, i_v, c_v, o_v0, o_v1, sem0, sem1):
    cid = lax.axis_index("c")
    sid = lax.axis_index("s")
    wid = sid * NC + cid
    base = wid * BPW

    pltpu.sync_copy(u_hbm.at[pl.ds(base * U * D, BPW * U * D)], u_v)
    pltpu.sync_copy(i_hbm.at[pl.ds(base * I * D, BPW * I * D)], i_v)
    pltpu.sync_copy(c_hbm.at[pl.ds(base * C * D, BPW * C * D)], c_v)

    def per_pair(g, carry):
        r0 = 2 * g
        r1 = r0 + 1

        @pl.when(g > 0)
        def _():
            pltpu.make_async_copy(o_v0, out_hbm.at[pl.ds(0, NR)], sem0).wait()

        _build_row(u_v, i_v, c_v, o_v0, r0)
        pltpu.async_copy(o_v0, out_hbm.at[pl.ds((base + r0) * NR, NR)], sem0)

        @pl.when(g > 0)
        def _():
            pltpu.make_async_copy(o_v1, out_hbm.at[pl.ds(0, NR)], sem1).wait()

        _build_row(u_v, i_v, c_v, o_v1, r1)
        pltpu.async_copy(o_v1, out_hbm.at[pl.ds((base + r1) * NR, NR)], sem1)
        return carry

    lax.fori_loop(0, BPW // 2, per_pair, 0)
    pltpu.make_async_copy(o_v0, out_hbm.at[pl.ds(0, NR)], sem0).wait()
    pltpu.make_async_copy(o_v1, out_hbm.at[pl.ds(0, NR)], sem1).wait()


_cross_call = functools.partial(
    pl.kernel,
    out_type=jax.ShapeDtypeStruct((B * NR,), jnp.float32),
    mesh=plsc.VectorSubcoreMesh(
        core_axis_name="c", subcore_axis_name="s",
        num_cores=NC, num_subcores=NS),
    scratch_types=[
        pltpu.VMEM((BPW * U * D,), jnp.float32),
        pltpu.VMEM((BPW * I * D,), jnp.float32),
        pltpu.VMEM((BPW * C * D,), jnp.float32),
        pltpu.VMEM((NR,), jnp.float32),
        pltpu.VMEM((NR,), jnp.float32),
        pltpu.SemaphoreType.DMA,
        pltpu.SemaphoreType.DMA,
    ],
)(_cross_body)


@jax.jit
def kernel(x_user, x_item, x_context):
    out = _cross_call(
        x_user.reshape(B * U * D),
        x_item.reshape(B * I * D),
        x_context.reshape(B * C * D),
    )
    return out.reshape(B, N, ROW)


# speedup vs baseline: 2.0789x; 1.4410x over previous
import functools

import jax
import jax.numpy as jnp
from jax import lax
from jax.experimental import pallas as pl
from jax.experimental.pallas import tpu as pltpu
from jax.experimental.pallas import tpu_sc as plsc

B = 4096
U, I, C = 8, 6, 3
D = 32
N = U * I * C
ROW = 3 * D
NR = N * ROW
NC, NS = 2, 16
NW = NC * NS
BPW = B // NW
L = 16


def _build_row(u_v, i_v, c_v, o_v, r):
    us = [u_v[pl.ds(r * U * D + t * L, L)] for t in range(U * D // L)]
    it = [i_v[pl.ds(r * I * D + t * L, L)] for t in range(I * D // L)]
    cs = [c_v[pl.ds(r * C * D + t * L, L)] for t in range(C * D // L)]
    for i in range(U):
        for j in range(I):
            for k in range(C):
                n = (i * I + j) * C + k
                o_v[n, pl.ds(0 * L, L)] = us[2 * i]
                o_v[n, pl.ds(1 * L, L)] = us[2 * i + 1]
                o_v[n, pl.ds(2 * L, L)] = it[2 * j]
                o_v[n, pl.ds(3 * L, L)] = it[2 * j + 1]
                o_v[n, pl.ds(4 * L, L)] = cs[2 * k]
                o_v[n, pl.ds(5 * L, L)] = cs[2 * k + 1]


def _cross_body(u_hbm, i_hbm, c_hbm, out_hbm,
                u_v, i_v, c_v, o_v0, o_v1, sem0, sem1):
    cid = lax.axis_index("c")
    sid = lax.axis_index("s")
    wid = sid * NC + cid
    base = wid * BPW

    pltpu.sync_copy(u_hbm.at[pl.ds(base * U * D, BPW * U * D)], u_v)
    pltpu.sync_copy(i_hbm.at[pl.ds(base * I * D, BPW * I * D)], i_v)
    pltpu.sync_copy(c_hbm.at[pl.ds(base * C * D, BPW * C * D)], c_v)

    def per_pair(g, carry):
        r0 = 2 * g
        r1 = r0 + 1

        @pl.when(g > 0)
        def _():
            pltpu.make_async_copy(o_v0, out_hbm.at[0], sem0).wait()

        _build_row(u_v, i_v, c_v, o_v0, r0)
        pltpu.async_copy(o_v0, out_hbm.at[base + r0], sem0)

        @pl.when(g > 0)
        def _():
            pltpu.make_async_copy(o_v1, out_hbm.at[0], sem1).wait()

        _build_row(u_v, i_v, c_v, o_v1, r1)
        pltpu.async_copy(o_v1, out_hbm.at[base + r1], sem1)
        return carry

    lax.fori_loop(0, BPW // 2, per_pair, 0)
    pltpu.make_async_copy(o_v0, out_hbm.at[0], sem0).wait()
    pltpu.make_async_copy(o_v1, out_hbm.at[0], sem1).wait()


_cross_call = functools.partial(
    pl.kernel,
    out_type=jax.ShapeDtypeStruct((B, N, ROW), jnp.float32),
    mesh=plsc.VectorSubcoreMesh(
        core_axis_name="c", subcore_axis_name="s",
        num_cores=NC, num_subcores=NS),
    compiler_params=pltpu.CompilerParams(use_tc_tiling_on_sc=True),
    scratch_types=[
        pltpu.VMEM((BPW * U * D,), jnp.float32),
        pltpu.VMEM((BPW * I * D,), jnp.float32),
        pltpu.VMEM((BPW * C * D,), jnp.float32),
        pltpu.VMEM((N, ROW), jnp.float32),
        pltpu.VMEM((N, ROW), jnp.float32),
        pltpu.SemaphoreType.DMA,
        pltpu.SemaphoreType.DMA,
    ],
)(_cross_body)


@jax.jit
def kernel(x_user, x_item, x_context):
    return _cross_call(
        x_user.reshape(B * U * D),
        x_item.reshape(B * I * D),
        x_context.reshape(B * C * D),
    )


# speedup vs baseline: 10.2319x; 4.9219x over previous
import functools

import jax
import jax.numpy as jnp
from jax import lax
from jax.experimental import pallas as pl
from jax.experimental.pallas import tpu as pltpu
from jax.experimental.pallas import tpu_sc as plsc

B = 4096
U, I, C = 8, 6, 3
D = 32
N = U * I * C
ROW = 3 * D
NC, NS = 2, 16
NW = NC * NS
BW = B // NW


def _cross_body(u_hbm, i_hbm, c_hbm, out_hbm, u_v, i_v, c_v, sem):
    cid = lax.axis_index("c")
    sid = lax.axis_index("s")
    wid = sid * NC + cid
    b0 = wid * BW

    pltpu.sync_copy(u_hbm.at[:, :, pl.ds(b0, BW)], u_v)
    pltpu.sync_copy(i_hbm.at[:, :, pl.ds(b0, BW)], i_v)
    pltpu.sync_copy(c_hbm.at[:, :, pl.ds(b0, BW)], c_v)

    for i in range(U):
        for j in range(I):
            for k in range(C):
                n = (i * I + j) * C + k
                pltpu.async_copy(
                    u_v.at[i], out_hbm.at[n, pl.ds(0, D), pl.ds(b0, BW)], sem)
                pltpu.async_copy(
                    i_v.at[j], out_hbm.at[n, pl.ds(D, D), pl.ds(b0, BW)], sem)
                pltpu.async_copy(
                    c_v.at[k], out_hbm.at[n, pl.ds(2 * D, D), pl.ds(b0, BW)],
                    sem)
    for _ in range(N):
        pltpu.make_async_copy(
            u_v.at[0], out_hbm.at[0, pl.ds(0, D), pl.ds(b0, BW)], sem).wait()
        pltpu.make_async_copy(
            i_v.at[0], out_hbm.at[0, pl.ds(D, D), pl.ds(b0, BW)], sem).wait()
        pltpu.make_async_copy(
            c_v.at[0], out_hbm.at[0, pl.ds(2 * D, D), pl.ds(b0, BW)],
            sem).wait()


_cross_call = functools.partial(
    pl.kernel,
    out_type=jax.ShapeDtypeStruct((N, ROW, B), jnp.float32),
    mesh=plsc.VectorSubcoreMesh(
        core_axis_name="c", subcore_axis_name="s",
        num_cores=NC, num_subcores=NS),
    compiler_params=pltpu.CompilerParams(use_tc_tiling_on_sc=True),
    scratch_types=[
        pltpu.VMEM((U, D, BW), jnp.float32),
        pltpu.VMEM((I, D, BW), jnp.float32),
        pltpu.VMEM((C, D, BW), jnp.float32),
        pltpu.SemaphoreType.DMA,
    ],
)(_cross_body)


@jax.jit
def kernel(x_user, x_item, x_context):
    out3 = _cross_call(
        jnp.transpose(x_user, (1, 2, 0)),
        jnp.transpose(x_item, (1, 2, 0)),
        jnp.transpose(x_context, (1, 2, 0)),
    )
    return jnp.transpose(out3, (2, 0, 1))


# speedup vs baseline: 10.4224x; 1.0186x over previous
import functools

import jax
import jax.numpy as jnp
from jax import lax
from jax.experimental import pallas as pl
from jax.experimental.pallas import tpu as pltpu
from jax.experimental.pallas import tpu_sc as plsc

B = 4096
U, I, C = 8, 6, 3
D = 32
N = U * I * C
ROW = 3 * D
NC, NS = 2, 16
NW = NC * NS
BW = B // NW


def _cross_body(u_hbm, i_hbm, c_hbm, out_hbm, u_v, i_v, c_v, sem):
    cid = lax.axis_index("c")
    sid = lax.axis_index("s")
    wid = sid * NC + cid
    b0 = wid * BW

    pltpu.async_copy(u_hbm.at[:, :, pl.ds(b0, BW)], u_v, sem)
    pltpu.async_copy(i_hbm.at[:, :, pl.ds(b0, BW)], i_v, sem)
    pltpu.async_copy(c_hbm.at[:, :, pl.ds(b0, BW)], c_v, sem)
    pltpu.make_async_copy(u_hbm.at[:, :, pl.ds(b0, BW)], u_v, sem).wait()
    pltpu.make_async_copy(i_hbm.at[:, :, pl.ds(b0, BW)], i_v, sem).wait()
    pltpu.make_async_copy(c_hbm.at[:, :, pl.ds(b0, BW)], c_v, sem).wait()

    for i in range(U):
        for j in range(I):
            for k in range(C):
                n = (i * I + j) * C + k
                pltpu.async_copy(
                    u_v.at[i], out_hbm.at[n, pl.ds(0, D), pl.ds(b0, BW)], sem)
                pltpu.async_copy(
                    i_v.at[j], out_hbm.at[n, pl.ds(D, D), pl.ds(b0, BW)], sem)
                pltpu.async_copy(
                    c_v.at[k], out_hbm.at[n, pl.ds(2 * D, D), pl.ds(b0, BW)],
                    sem)
    for _ in range(N):
        pltpu.make_async_copy(
            u_v.at[0], out_hbm.at[0, pl.ds(0, D), pl.ds(b0, BW)], sem).wait()
        pltpu.make_async_copy(
            i_v.at[0], out_hbm.at[0, pl.ds(D, D), pl.ds(b0, BW)], sem).wait()
        pltpu.make_async_copy(
            c_v.at[0], out_hbm.at[0, pl.ds(2 * D, D), pl.ds(b0, BW)],
            sem).wait()


_cross_call = functools.partial(
    pl.kernel,
    out_type=jax.ShapeDtypeStruct((N, ROW, B), jnp.float32),
    mesh=plsc.VectorSubcoreMesh(
        core_axis_name="c", subcore_axis_name="s",
        num_cores=NC, num_subcores=NS),
    compiler_params=pltpu.CompilerParams(use_tc_tiling_on_sc=True),
    scratch_types=[
        pltpu.VMEM((U, D, BW), jnp.float32),
        pltpu.VMEM((I, D, BW), jnp.float32),
        pltpu.VMEM((C, D, BW), jnp.float32),
        pltpu.SemaphoreType.DMA,
    ],
)(_cross_body)


@jax.jit
def kernel(x_user, x_item, x_context):
    out3 = _cross_call(
        jnp.transpose(x_user, (1, 2, 0)),
        jnp.transpose(x_item, (1, 2, 0)),
        jnp.transpose(x_context, (1, 2, 0)),
    )
    return jnp.transpose(out3, (2, 0, 1))


# speedup vs baseline: 10.4826x; 1.0058x over previous
import functools

import jax
import jax.numpy as jnp
from jax import lax
from jax.experimental import pallas as pl
from jax.experimental.pallas import tpu as pltpu
from jax.experimental.pallas import tpu_sc as plsc

B = 4096
U, I, C = 8, 6, 3
D = 32
N = U * I * C
ROW = 3 * D
NC, NS = 2, 16
NW = NC * NS
BW = B // NW


def _cross_body(u_hbm, i_hbm, c_hbm, out_hbm, u_v, i_v, c_v, sem, sem_in):
    cid = lax.axis_index("c")
    sid = lax.axis_index("s")
    wid = sid * NC + cid
    b0 = wid * BW

    pltpu.async_copy(u_hbm.at[:, :, pl.ds(b0, BW)], u_v, sem_in)
    pltpu.async_copy(i_hbm.at[:, :, pl.ds(b0, BW)], i_v, sem_in)
    pltpu.async_copy(c_hbm.at[:, :, pl.ds(b0, BW)], c_v, sem_in)

    pltpu.make_async_copy(u_hbm.at[:, :, pl.ds(b0, BW)], u_v, sem_in).wait()
    for i in range(U):
        for m in range(I * C):
            n = i * I * C + m
            pltpu.async_copy(
                u_v.at[i], out_hbm.at[n, pl.ds(0, D), pl.ds(b0, BW)], sem)
    pltpu.make_async_copy(i_hbm.at[:, :, pl.ds(b0, BW)], i_v, sem_in).wait()
    for j in range(I):
        for i in range(U):
            for k in range(C):
                n = (i * I + j) * C + k
                pltpu.async_copy(
                    i_v.at[j], out_hbm.at[n, pl.ds(D, D), pl.ds(b0, BW)], sem)
    pltpu.make_async_copy(c_hbm.at[:, :, pl.ds(b0, BW)], c_v, sem_in).wait()
    for k in range(C):
        for m in range(U * I):
            n = m * C + k
            pltpu.async_copy(
                c_v.at[k], out_hbm.at[n, pl.ds(2 * D, D), pl.ds(b0, BW)], sem)
    for _ in range(N):
        pltpu.make_async_copy(
            u_v.at[0], out_hbm.at[0, pl.ds(0, D), pl.ds(b0, BW)], sem).wait()
        pltpu.make_async_copy(
            i_v.at[0], out_hbm.at[0, pl.ds(D, D), pl.ds(b0, BW)], sem).wait()
        pltpu.make_async_copy(
            c_v.at[0], out_hbm.at[0, pl.ds(2 * D, D), pl.ds(b0, BW)],
            sem).wait()


_cross_call = functools.partial(
    pl.kernel,
    out_type=jax.ShapeDtypeStruct((N, ROW, B), jnp.float32),
    mesh=plsc.VectorSubcoreMesh(
        core_axis_name="c", subcore_axis_name="s",
        num_cores=NC, num_subcores=NS),
    compiler_params=pltpu.CompilerParams(use_tc_tiling_on_sc=True),
    scratch_types=[
        pltpu.VMEM((U, D, BW), jnp.float32),
        pltpu.VMEM((I, D, BW), jnp.float32),
        pltpu.VMEM((C, D, BW), jnp.float32),
        pltpu.SemaphoreType.DMA,
        pltpu.SemaphoreType.DMA,
    ],
)(_cross_body)


@jax.jit
def kernel(x_user, x_item, x_context):
    out3 = _cross_call(
        jnp.transpose(x_user, (1, 2, 0)),
        jnp.transpose(x_item, (1, 2, 0)),
        jnp.transpose(x_context, (1, 2, 0)),
    )
    return jnp.transpose(out3, (2, 0, 1))


# speedup vs baseline: 10.5119x; 1.0028x over previous
import functools

import jax
import jax.numpy as jnp
from jax import lax
from jax.experimental import pallas as pl
from jax.experimental.pallas import tpu as pltpu
from jax.experimental.pallas import tpu_sc as plsc

B = 4096
U, I, C = 8, 6, 3
D = 32
N = U * I * C
ROW = 3 * D
NC, NS = 2, 16
UH = U // NC
BW = B // NS


def _cross_body(u_hbm, i_hbm, c_hbm, out_hbm, u_v, i_v, c_v, sem, sem_in):
    cid = lax.axis_index("c")
    sid = lax.axis_index("s")
    b0 = sid * BW
    u0 = cid * UH
    n0 = u0 * I * C

    pltpu.async_copy(u_hbm.at[pl.ds(u0, UH), :, pl.ds(b0, BW)], u_v, sem_in)
    pltpu.async_copy(i_hbm.at[:, :, pl.ds(b0, BW)], i_v, sem_in)
    pltpu.async_copy(c_hbm.at[:, :, pl.ds(b0, BW)], c_v, sem_in)

    pltpu.make_async_copy(
        u_hbm.at[pl.ds(u0, UH), :, pl.ds(b0, BW)], u_v, sem_in).wait()
    for i in range(UH):
        for m in range(I * C):
            n = n0 + i * I * C + m
            pltpu.async_copy(
                u_v.at[i], out_hbm.at[n, pl.ds(0, D), pl.ds(b0, BW)], sem)
    pltpu.make_async_copy(i_hbm.at[:, :, pl.ds(b0, BW)], i_v, sem_in).wait()
    for j in range(I):
        for i in range(UH):
            for k in range(C):
                n = n0 + (i * I + j) * C + k
                pltpu.async_copy(
                    i_v.at[j], out_hbm.at[n, pl.ds(D, D), pl.ds(b0, BW)], sem)
    pltpu.make_async_copy(c_hbm.at[:, :, pl.ds(b0, BW)], c_v, sem_in).wait()
    for k in range(C):
        for m in range(UH * I):
            n = n0 + m * C + k
            pltpu.async_copy(
                c_v.at[k], out_hbm.at[n, pl.ds(2 * D, D), pl.ds(b0, BW)], sem)
    for _ in range(UH * I * C):
        pltpu.make_async_copy(
            u_v.at[0], out_hbm.at[0, pl.ds(0, D), pl.ds(b0, BW)], sem).wait()
        pltpu.make_async_copy(
            i_v.at[0], out_hbm.at[0, pl.ds(D, D), pl.ds(b0, BW)], sem).wait()
        pltpu.make_async_copy(
            c_v.at[0], out_hbm.at[0, pl.ds(2 * D, D), pl.ds(b0, BW)],
            sem).wait()


_cross_call = functools.partial(
    pl.kernel,
    out_type=jax.ShapeDtypeStruct((N, ROW, B), jnp.float32),
    mesh=plsc.VectorSubcoreMesh(
        core_axis_name="c", subcore_axis_name="s",
        num_cores=NC, num_subcores=NS),
    compiler_params=pltpu.CompilerParams(use_tc_tiling_on_sc=True),
    scratch_types=[
        pltpu.VMEM((UH, D, BW), jnp.float32),
        pltpu.VMEM((I, D, BW), jnp.float32),
        pltpu.VMEM((C, D, BW), jnp.float32),
        pltpu.SemaphoreType.DMA,
        pltpu.SemaphoreType.DMA,
    ],
)(_cross_body)


@jax.jit
def kernel(x_user, x_item, x_context):
    out3 = _cross_call(
        jnp.transpose(x_user, (1, 2, 0)),
        jnp.transpose(x_item, (1, 2, 0)),
        jnp.transpose(x_context, (1, 2, 0)),
    )
    return jnp.transpose(out3, (2, 0, 1))
